# baseline (speedup 1.0000x reference)
import functools

import jax
import jax.numpy as jnp
from jax import lax
from jax.experimental import pallas as pl
from jax.experimental.pallas import tpu as pltpu
from jax.experimental.pallas import tpu_sc as plsc

N = 10000
E = 320000
H = 128
G = 64

NC = 2
NS = 16
NW = NC * NS
EPW = E // NW
CH = 128
NCHUNK = 80
EPT = NCHUNK * CH
TRASH = N
NPT = 624
NTAIL = N - NPT * NS

BN = 2000
GRID = N // BN


def _agg_body(h_hbm, srcs_hbm, dsts_hbm, out_hbm,
              src_v, rows_v, dst_v, agg_sh, sem):
    c = lax.axis_index("c")
    s = lax.axis_index("s")
    row0 = s * NPT

    pltpu.sync_copy(h_hbm.at[pl.ds(row0, NPT)], agg_sh.at[pl.ds(row0, NPT)])

    @pl.when(s == NS - 1)
    def _():
        pltpu.sync_copy(h_hbm.at[pl.ds(NPT * NS, NTAIL)],
                        agg_sh.at[pl.ds(NPT * NS, NTAIL)])

    plsc.subcore_barrier()

    ebase = (c * NS + s) * EPT

    def chunk(k, carry):
        base = ebase + k * CH
        pltpu.sync_copy(srcs_hbm.at[pl.ds(base, CH)], src_v)
        pltpu.async_copy(h_hbm.at[src_v], rows_v, sem).wait()
        pltpu.sync_copy(dsts_hbm.at[pl.ds(base, CH)], dst_v)
        pltpu.sync_copy(rows_v, agg_sh.at[dst_v], add=True)
        return carry

    lax.fori_loop(0, NCHUNK, chunk, 0)

    plsc.subcore_barrier()

    pltpu.sync_copy(agg_sh.at[pl.ds(row0, NPT)],
                    out_hbm.at[pl.ds(c * N + row0, NPT)])

    @pl.when(s == NS - 1)
    def _():
        pltpu.sync_copy(agg_sh.at[pl.ds(NPT * NS, NTAIL)],
                        out_hbm.at[pl.ds(c * N + NPT * NS, NTAIL)])


@functools.cache
def _get_agg_call():
    return pl.kernel(
        _agg_body,
        out_type=jax.ShapeDtypeStruct((2 * N, H), jnp.float32),
        mesh=plsc.VectorSubcoreMesh(core_axis_name="c", subcore_axis_name="s",
                                    num_cores=NC, num_subcores=NS),
        scratch_types=[
            pltpu.VMEM((CH,), jnp.int32),
            pltpu.VMEM((CH, H), jnp.float32),
            pltpu.VMEM((CH,), jnp.int32),
            pltpu.VMEM_SHARED((N + 8, H), jnp.float32),
            pltpu.SemaphoreType.DMA,
        ],
        name="gin_edge_agg_sc",
    )


def _agg_call(h, src_s, dst_s):
    return _get_agg_call()(h, src_s, dst_s)


def _mlp_body(a0, a1, h, w1, b1, w2, b2, o):
    z = a0[...] + a1[...] - h[...]
    z = lax.dot(z, w1[...], preferred_element_type=jnp.float32) + b1[...]
    z = jnp.maximum(z, 0.0)
    z = lax.dot(z, w2[...], preferred_element_type=jnp.float32) + b2[...]
    o[...] = jnp.maximum(z, 0.0)


def _tc_mlp(agg2, h, w1, b1, w2, b2):
    return pl.pallas_call(
        _mlp_body,
        grid=(GRID,),
        in_specs=[
            pl.BlockSpec((BN, H), lambda i: (i, 0)),
            pl.BlockSpec((BN, H), lambda i: (i + GRID, 0)),
            pl.BlockSpec((BN, H), lambda i: (i, 0)),
            pl.BlockSpec((H, H), lambda i: (0, 0)),
            pl.BlockSpec((1, H), lambda i: (0, 0)),
            pl.BlockSpec((H, H), lambda i: (0, 0)),
            pl.BlockSpec((1, H), lambda i: (0, 0)),
        ],
        out_specs=pl.BlockSpec((BN, H), lambda i: (i, 0)),
        out_shape=jax.ShapeDtypeStruct((N, H), jnp.float32),
        name="gin_mlp_tc",
    )(agg2, agg2, h, w1, b1, w2, b2)


def _mlp_pool_body(a0, a1, h, w1, b1, w2, b2, bt, hw, hb, o, sums, cnts):
    i = pl.program_id(0)

    z = a0[...] + a1[...] - h[...]
    z = lax.dot(z, w1[...], preferred_element_type=jnp.float32) + b1[...]
    z = jnp.maximum(z, 0.0)
    z = lax.dot(z, w2[...], preferred_element_type=jnp.float32) + b2[...]
    z = jnp.maximum(z, 0.0)

    @pl.when(i == 0)
    def _():
        sums[...] = jnp.zeros_like(sums)
        cnts[...] = jnp.zeros_like(cnts)

    onehot = (bt[...] == lax.broadcasted_iota(jnp.int32, (1, G), 1))
    onehot = onehot.astype(jnp.float32)
    sums[...] += lax.dot_general(onehot, z, (((0,), (0,)), ((), ())),
                                 precision=lax.Precision.HIGHEST,
                                 preferred_element_type=jnp.float32)
    ones = jnp.ones((BN, 1), jnp.float32)
    cnts[...] += lax.dot_general(onehot, ones, (((0,), (0,)), ((), ())),
                                 precision=lax.Precision.HIGHEST,
                                 preferred_element_type=jnp.float32)

    @pl.when(i == GRID - 1)
    def _():
        pooled = sums[...] / jnp.maximum(cnts[...], 1.0)
        o[...] = lax.dot(pooled, hw[...],
                         preferred_element_type=jnp.float32) + hb[...]


def _tc_mlp_pool(agg2, h, w1, b1, w2, b2, batch2d, head_w, head_b2d):
    return pl.pallas_call(
        _mlp_pool_body,
        grid=(GRID,),
        in_specs=[
            pl.BlockSpec((BN, H), lambda i: (i, 0)),
            pl.BlockSpec((BN, H), lambda i: (i + GRID, 0)),
            pl.BlockSpec((BN, H), lambda i: (i, 0)),
            pl.BlockSpec((H, H), lambda i: (0, 0)),
            pl.BlockSpec((1, H), lambda i: (0, 0)),
            pl.BlockSpec((H, H), lambda i: (0, 0)),
            pl.BlockSpec((1, H), lambda i: (0, 0)),
            pl.BlockSpec((BN, 1), lambda i: (i, 0)),
            pl.BlockSpec((H, 1), lambda i: (0, 0)),
            pl.BlockSpec((1, 1), lambda i: (0, 0)),
        ],
        out_specs=pl.BlockSpec((G, 1), lambda i: (0, 0)),
        out_shape=jax.ShapeDtypeStruct((G, 1), jnp.float32),
        scratch_shapes=[
            pltpu.VMEM((G, H), jnp.float32),
            pltpu.VMEM((G, 1), jnp.float32),
        ],
        name="gin_mlp_pool_tc",
    )(agg2, agg2, h, w1, b1, w2, b2, batch2d, head_w, head_b2d)


def kernel(x, edge_index, batch, params):
    src = edge_index[0]
    dst = edge_index[1]

    src_s = jnp.pad(src.reshape(NW, EPW),
                    ((0, 0), (0, EPT - EPW))).reshape(-1)
    dst_s = jnp.pad(dst.reshape(NW, EPW), ((0, 0), (0, EPT - EPW)),
                    constant_values=TRASH).reshape(-1)

    h = x
    layers = params["layers"]
    for (w1, b1, w2, b2) in layers[:-1]:
        agg2 = _agg_call(h, src_s, dst_s)
        h = _tc_mlp(agg2, h, w1, b1.reshape(1, H), w2, b2.reshape(1, H))
    (w1, b1, w2, b2) = layers[-1]
    agg2 = _agg_call(h, src_s, dst_s)
    return _tc_mlp_pool(agg2, h, w1, b1.reshape(1, H), w2, b2.reshape(1, H),
                        batch.reshape(N, 1), params["head_W"],
                        params["head_b"].reshape(1, 1))


# speedup vs baseline: 1.6568x; 1.6568x over previous
import functools

import jax
import jax.numpy as jnp
from jax import lax
from jax.experimental import pallas as pl
from jax.experimental.pallas import tpu as pltpu
from jax.experimental.pallas import tpu_sc as plsc

N = 10000
E = 320000
H = 128
G = 64

NC = 2
NS = 16
NW = NC * NS
EPW = E // NW
CH = 80
NCHUNK = EPW // CH
NPT = 624
NTAIL = N - NPT * NS

BN = 2000
GRID = N // BN


def _agg_body(h_hbm, srcs_hbm, dsts_hbm, out_hbm,
              src_v, rows_v, dst_v, agg_sh, sem):
    c = lax.axis_index("c")
    s = lax.axis_index("s")
    row0 = s * NPT

    pltpu.sync_copy(h_hbm.at[pl.ds(row0, NPT)], agg_sh.at[pl.ds(row0, NPT)])

    @pl.when(s == NS - 1)
    def _():
        pltpu.sync_copy(h_hbm.at[pl.ds(NPT * NS, NTAIL)],
                        agg_sh.at[pl.ds(NPT * NS, NTAIL)])

    plsc.subcore_barrier()

    ebase = (c * NS + s) * EPW

    def chunk(k, carry):
        base = ebase + k * CH
        pltpu.sync_copy(srcs_hbm.at[pl.ds(base, CH)], src_v)
        pltpu.async_copy(h_hbm.at[src_v], rows_v, sem).wait()
        pltpu.sync_copy(dsts_hbm.at[pl.ds(base, CH)], dst_v)
        pltpu.sync_copy(rows_v, agg_sh.at[dst_v], add=True)
        return carry

    lax.fori_loop(0, NCHUNK, chunk, 0)

    plsc.subcore_barrier()

    pltpu.sync_copy(agg_sh.at[pl.ds(row0, NPT)],
                    out_hbm.at[pl.ds(c * N + row0, NPT)])

    @pl.when(s == NS - 1)
    def _():
        pltpu.sync_copy(agg_sh.at[pl.ds(NPT * NS, NTAIL)],
                        out_hbm.at[pl.ds(c * N + NPT * NS, NTAIL)])


@functools.cache
def _get_agg_call():
    return pl.kernel(
        _agg_body,
        out_type=jax.ShapeDtypeStruct((2 * N, H), jnp.float32),
        mesh=plsc.VectorSubcoreMesh(core_axis_name="c", subcore_axis_name="s",
                                    num_cores=NC, num_subcores=NS),
        scratch_types=[
            pltpu.VMEM((CH,), jnp.int32),
            pltpu.VMEM((CH, H), jnp.float32),
            pltpu.VMEM((CH,), jnp.int32),
            pltpu.VMEM_SHARED((N, H), jnp.float32),
            pltpu.SemaphoreType.DMA,
        ],
        name="gin_edge_agg_sc",
    )


def _agg_call(h, src_s, dst_s):
    return _get_agg_call()(h, src_s, dst_s)


def _mlp_body(a0, a1, h, w1, b1, w2, b2, o):
    z = a0[...] + a1[...] - h[...]
    z = lax.dot(z, w1[...], preferred_element_type=jnp.float32) + b1[...]
    z = jnp.maximum(z, 0.0)
    z = lax.dot(z, w2[...], preferred_element_type=jnp.float32) + b2[...]
    o[...] = jnp.maximum(z, 0.0)


def _tc_mlp(agg2, h, w1, b1, w2, b2):
    return pl.pallas_call(
        _mlp_body,
        grid=(GRID,),
        in_specs=[
            pl.BlockSpec((BN, H), lambda i: (i, 0)),
            pl.BlockSpec((BN, H), lambda i: (i + GRID, 0)),
            pl.BlockSpec((BN, H), lambda i: (i, 0)),
            pl.BlockSpec((H, H), lambda i: (0, 0)),
            pl.BlockSpec((1, H), lambda i: (0, 0)),
            pl.BlockSpec((H, H), lambda i: (0, 0)),
            pl.BlockSpec((1, H), lambda i: (0, 0)),
        ],
        out_specs=pl.BlockSpec((BN, H), lambda i: (i, 0)),
        out_shape=jax.ShapeDtypeStruct((N, H), jnp.float32),
        name="gin_mlp_tc",
    )(agg2, agg2, h, w1, b1, w2, b2)


def _mlp_pool_body(a0, a1, h, w1, b1, w2, b2, bt, hw, hb, o, sums, cnts):
    i = pl.program_id(0)

    z = a0[...] + a1[...] - h[...]
    z = lax.dot(z, w1[...], preferred_element_type=jnp.float32) + b1[...]
    z = jnp.maximum(z, 0.0)
    z = lax.dot(z, w2[...], preferred_element_type=jnp.float32) + b2[...]
    z = jnp.maximum(z, 0.0)

    @pl.when(i == 0)
    def _():
        sums[...] = jnp.zeros_like(sums)
        cnts[...] = jnp.zeros_like(cnts)

    onehot = (bt[...] == lax.broadcasted_iota(jnp.int32, (1, G), 1))
    onehot = onehot.astype(jnp.float32)
    sums[...] += lax.dot_general(onehot, z, (((0,), (0,)), ((), ())),
                                 precision=lax.Precision.HIGHEST,
                                 preferred_element_type=jnp.float32)
    ones = jnp.ones((BN, 1), jnp.float32)
    cnts[...] += lax.dot_general(onehot, ones, (((0,), (0,)), ((), ())),
                                 precision=lax.Precision.HIGHEST,
                                 preferred_element_type=jnp.float32)

    @pl.when(i == GRID - 1)
    def _():
        pooled = sums[...] / jnp.maximum(cnts[...], 1.0)
        o[...] = lax.dot(pooled, hw[...],
                         preferred_element_type=jnp.float32) + hb[...]


def _tc_mlp_pool(agg2, h, w1, b1, w2, b2, batch2d, head_w, head_b2d):
    return pl.pallas_call(
        _mlp_pool_body,
        grid=(GRID,),
        in_specs=[
            pl.BlockSpec((BN, H), lambda i: (i, 0)),
            pl.BlockSpec((BN, H), lambda i: (i + GRID, 0)),
            pl.BlockSpec((BN, H), lambda i: (i, 0)),
            pl.BlockSpec((H, H), lambda i: (0, 0)),
            pl.BlockSpec((1, H), lambda i: (0, 0)),
            pl.BlockSpec((H, H), lambda i: (0, 0)),
            pl.BlockSpec((1, H), lambda i: (0, 0)),
            pl.BlockSpec((BN, 1), lambda i: (i, 0)),
            pl.BlockSpec((H, 1), lambda i: (0, 0)),
            pl.BlockSpec((1, 1), lambda i: (0, 0)),
        ],
        out_specs=pl.BlockSpec((G, 1), lambda i: (0, 0)),
        out_shape=jax.ShapeDtypeStruct((G, 1), jnp.float32),
        scratch_shapes=[
            pltpu.VMEM((G, H), jnp.float32),
            pltpu.VMEM((G, 1), jnp.float32),
        ],
        name="gin_mlp_pool_tc",
    )(agg2, agg2, h, w1, b1, w2, b2, batch2d, head_w, head_b2d)


def kernel(x, edge_index, batch, params):
    src = edge_index[0]
    dst = edge_index[1]

    src_s = src
    dst_s = dst

    h = x
    layers = params["layers"]
    for (w1, b1, w2, b2) in layers[:-1]:
        agg2 = _agg_call(h, src_s, dst_s)
        h = _tc_mlp(agg2, h, w1, b1.reshape(1, H), w2, b2.reshape(1, H))
    (w1, b1, w2, b2) = layers[-1]
    agg2 = _agg_call(h, src_s, dst_s)
    return _tc_mlp_pool(agg2, h, w1, b1.reshape(1, H), w2, b2.reshape(1, H),
                        batch.reshape(N, 1), params["head_W"],
                        params["head_b"].reshape(1, 1))
